# Initial kernel scaffold; baseline (speedup 1.0000x reference)
import jax
import jax.numpy as jnp
from jax.experimental import pallas as pl


def kernel(latent_z, gamma, bias, segment_ids, sparse_i, sparse_j):
    raise NotImplementedError("write your pallas kernel here")


# speedup vs baseline: 76.2343x; 76.2343x over previous
import functools

import jax
import jax.numpy as jnp
from jax import lax
from jax.experimental import pallas as pl
from jax.experimental.pallas import tpu as pltpu
from jax.experimental.pallas import tpu_sc as plsc

def _sc_sqrt(x):
    xi = plsc.bitcast(x, jnp.int32)
    ri = jnp.int32(0x5F3759DF) - lax.shift_right_logical(xi, jnp.ones_like(xi))
    r = plsc.bitcast(ri, jnp.float32)
    half_x = 0.5 * x
    for _ in range(3):
        r = r * (1.5 - half_x * r * r)
    return x * r


_K = 1024
_NW = 32
_G = 128
_B = 1024


def _sc_edge_call(aug, idx_rows, T):
    mesh = plsc.VectorSubcoreMesh(core_axis_name="c", subcore_axis_name="s")

    @functools.partial(
        pl.kernel,
        mesh=mesh,
        compiler_params=pltpu.CompilerParams(
            needs_layout_passes=False, use_tc_tiling_on_sc=False),
        out_type=jax.ShapeDtypeStruct((_NW, 16), jnp.float32),
        scratch_types=[
            pltpu.VMEM((2, _G), jnp.int32),
            pltpu.VMEM((2, _G), jnp.int32),
            pltpu.VMEM((_G, 16), jnp.float32),
            pltpu.VMEM((_G, 16), jnp.float32),
            pltpu.VMEM((_G, 16), jnp.float32),
            pltpu.VMEM((_G, 16), jnp.float32),
            pltpu.VMEM((16,), jnp.float32),
            pltpu.SemaphoreType.DMA,
            pltpu.SemaphoreType.DMA,
            pltpu.SemaphoreType.DMA,
            pltpu.SemaphoreType.DMA,
            pltpu.SemaphoreType.DMA,
            pltpu.SemaphoreType.DMA,
        ],
    )
    def edge_kernel(aug_hbm, idx_hbm, out_hbm, idx0, idx1, zi0, zi1,
                    zj0, zj1, accv, sx0, sx1, si0, si1, sj0, sj1):
        idxb = (idx0, idx1)
        zib = (zi0, zi1)
        zjb = (zj0, zj1)
        sx = (sx0, sx1)
        si = (si0, si1)
        sj = (sj0, sj1)
        wid = lax.axis_index("s") * 2 + lax.axis_index("c")
        row0 = wid * T

        def idx_issue(t, b):
            return pltpu.async_copy(idx_hbm.at[row0 + t], idxb[b], sx[b])

        def idx_wait(t, b):
            pltpu.make_async_copy(idx_hbm.at[row0 + t], idxb[b],
                                  sx[b]).wait()

        def gather_issue(b):
            pltpu.async_copy(aug_hbm.at[idxb[b].at[0]], zib[b], si[b])
            pltpu.async_copy(aug_hbm.at[idxb[b].at[1]], zjb[b], sj[b])

        def gather_wait(b):
            pltpu.make_async_copy(aug_hbm.at[idxb[b].at[0]], zib[b],
                                  si[b]).wait()
            pltpu.make_async_copy(aug_hbm.at[idxb[b].at[1]], zjb[b],
                                  sj[b]).wait()

        def compute(b, acc):
            for u in range(_G // 16):
                e = u * 16 + lax.iota(jnp.int32, 16)
                d2 = jnp.full((16,), 1e-8, jnp.float32)
                for d in range(8):
                    col = jnp.full((16,), d, jnp.int32)
                    ai = plsc.load_gather(zib[b], [e, col])
                    aj = plsc.load_gather(zjb[b], [e, col])
                    t_ = ai - aj
                    d2 = d2 + t_ * t_
                col8 = jnp.full((16,), 8, jnp.int32)
                gi = plsc.load_gather(zib[b], [e, col8])
                gj = plsc.load_gather(zjb[b], [e, col8])
                acc = acc + (gi + gj - _sc_sqrt(d2))
            return acc

        h0 = idx_issue(0, 0)
        idx_issue(1, 1)
        h0.wait()
        gather_issue(0)

        def one_step(t, b, acc):
            gather_wait(b)

            @pl.when(t + 2 < T)
            def _():
                idx_issue(t + 2, b)

            @pl.when(t + 1 < T)
            def _():
                idx_wait(t + 1, 1 - b)
                gather_issue(1 - b)

            return compute(b, acc)

        def pair(k, acc):
            t0 = 2 * k
            acc = one_step(t0, 0, acc)
            acc = one_step(t0 + 1, 1, acc)
            return acc

        acc = lax.fori_loop(0, T // 2, pair,
                            jnp.zeros((16,), jnp.float32))
        accv[...] = acc
        pltpu.sync_copy(accv, out_hbm.at[wid])

    return edge_kernel(aug, idx_rows)


def _tc_body(n_steps, n_edges, seg_ref, z_ref, g_ref, bias_ref, part_ref,
             out_ref, stats):
    i = pl.program_id(0)

    @pl.when(i == 0)
    def _():
        stats[...] = jnp.zeros((_K, 16), jnp.float32)

    seg = seg_ref[...]
    iota_k = lax.broadcasted_iota(jnp.int32, (_K, _B), 0)
    oh = jnp.where(iota_k == seg, 1.0, 0.0).astype(jnp.float32)
    eg = jnp.exp(g_ref[...])
    ones = jnp.ones((_B, 1), jnp.float32)
    zeros6 = jnp.zeros((_B, 6), jnp.float32)
    payload = jnp.concatenate([z_ref[...], ones, eg, zeros6], axis=1)
    stats[...] += lax.dot_general(
        oh, payload, (((1,), (0,)), ((), ())),
        preferred_element_type=jnp.float32,
        precision=lax.Precision.HIGHEST)

    @pl.when(i == n_steps - 1)
    def _():
        st = stats[...]
        counts = st[:, 8:9]
        s_col = st[:, 9:10]
        cm = st[:, 0:8] / jnp.maximum(counts, 1.0)
        cc = cm * cm
        n_col = jnp.sum(cc, axis=1, keepdims=True)
        ones8 = jnp.ones((1, 8), jnp.float32)
        n_row = lax.dot_general(
            ones8, cc, (((1,), (1,)), ((), ())),
            preferred_element_type=jnp.float32,
            precision=lax.Precision.HIGHEST)
        e9 = jnp.where(
            lax.broadcasted_iota(jnp.int32, (1, 16), 1) == 9, 1.0, 0.0
        ).astype(jnp.float32)
        s_row = lax.dot_general(
            e9, st, (((1,), (1,)), ((), ())),
            preferred_element_type=jnp.float32,
            precision=lax.Precision.HIGHEST)
        bias = bias_ref[0, 0]
        total = jnp.float32(0.0)
        for rb in range(_K // 128):
            r0 = rb * 128
            cr = cm[r0:r0 + 128, :]
            g_mat = lax.dot_general(
                cr, cm, (((1,), (1,)), ((), ())),
                preferred_element_type=jnp.float32,
                precision=lax.Precision.HIGHEST)
            d2 = jnp.maximum(
                n_col[r0:r0 + 128, :] + n_row - 2.0 * g_mat, 0.0) + 1e-8
            kx = jnp.exp(bias - jnp.sqrt(d2))
            w = s_col[r0:r0 + 128, :] * s_row
            row_id = r0 + lax.broadcasted_iota(jnp.int32, (128, _K), 0)
            col_id = lax.broadcasted_iota(jnp.int32, (128, _K), 1)
            total += jnp.sum(jnp.where(col_id > row_id, kx * w, 0.0))
        link = (jnp.float32(n_edges) * bias + jnp.sum(part_ref[...]))
        out_ref[...] = jnp.reshape(link - total, (1, 1))


def kernel(latent_z, gamma, bias, segment_ids, sparse_i, sparse_j):
    n = latent_z.shape[0]
    e = sparse_i.shape[0]
    z = latent_z.astype(jnp.float32)
    g = gamma.astype(jnp.float32)

    aug = jnp.concatenate(
        [z, g[:, None], jnp.zeros((n, 7), jnp.float32)], axis=1)
    aug = jnp.concatenate([aug, jnp.zeros((8, 16), jnp.float32)], axis=0)
    grp = _NW * _G
    ep = ((e + grp - 1) // grp) * grp
    pad = ep - e
    t_per_w = ep // grp
    assert t_per_w % 2 == 0 and t_per_w >= 2
    si = jnp.concatenate(
        [sparse_i.astype(jnp.int32), jnp.full((pad,), n, jnp.int32)])
    sj = jnp.concatenate(
        [sparse_j.astype(jnp.int32), jnp.full((pad,), n, jnp.int32)])
    idx_rows = jnp.stack(
        [si.reshape(ep // _G, _G), sj.reshape(ep // _G, _G)], axis=1)

    partials = _sc_edge_call(aug, idx_rows, t_per_w)

    pad_fix = jnp.float32(pad) * jnp.float32(1e-8) ** 0.5

    n_pad = ((n + _B - 1) // _B) * _B
    n_steps = n_pad // _B
    seg_row = jnp.concatenate(
        [segment_ids.astype(jnp.int32),
         jnp.full((n_pad - n,), _K, jnp.int32)]).reshape(1, n_pad)
    z_pad = jnp.concatenate(
        [z, jnp.zeros((n_pad - n, 8), jnp.float32)], axis=0)
    g_pad = jnp.concatenate(
        [g, jnp.zeros((n_pad - n,), jnp.float32)]).reshape(n_pad, 1)
    bias2 = bias.astype(jnp.float32).reshape(1, 1)

    out = pl.pallas_call(
        functools.partial(_tc_body, n_steps, e),
        grid=(n_steps,),
        in_specs=[
            pl.BlockSpec((1, _B), lambda i: (0, i)),
            pl.BlockSpec((_B, 8), lambda i: (i, 0)),
            pl.BlockSpec((_B, 1), lambda i: (i, 0)),
            pl.BlockSpec((1, 1), lambda i: (0, 0)),
            pl.BlockSpec((_NW, 16), lambda i: (0, 0)),
        ],
        out_specs=pl.BlockSpec((1, 1), lambda i: (0, 0)),
        out_shape=jax.ShapeDtypeStruct((1, 1), jnp.float32),
        scratch_shapes=[pltpu.VMEM((_K, 16), jnp.float32)],
    )(seg_row, z_pad, g_pad, bias2, partials)

    return out[0, 0] + pad_fix


# speedup vs baseline: 94.1580x; 1.2351x over previous
import functools

import jax
import jax.numpy as jnp
from jax import lax
from jax.experimental import pallas as pl
from jax.experimental.pallas import tpu as pltpu
from jax.experimental.pallas import tpu_sc as plsc

def _sc_sqrt(x):
    xi = plsc.bitcast(x, jnp.int32)
    ri = jnp.int32(0x5F3759DF) - lax.shift_right_logical(xi, jnp.ones_like(xi))
    r = plsc.bitcast(ri, jnp.float32)
    half_x = 0.5 * x
    for _ in range(3):
        r = r * (1.5 - half_x * r * r)
    return x * r


_K = 1024
_NW = 32
_G = 128
_B = 1024


def _sc_edge_call(aug, idx_rows, T):
    n_rows = aug.shape[0]
    mesh = plsc.VectorSubcoreMesh(core_axis_name="c", subcore_axis_name="s")

    @functools.partial(
        pl.kernel,
        mesh=mesh,
        compiler_params=pltpu.CompilerParams(
            needs_layout_passes=False, use_tc_tiling_on_sc=False),
        out_type=jax.ShapeDtypeStruct((_NW, 16), jnp.float32),
        scratch_types=[
            pltpu.VMEM((2, _G), jnp.int32),
            pltpu.VMEM((2, _G), jnp.int32),
            pltpu.VMEM((_G, 16), jnp.float32),
            pltpu.VMEM((_G, 16), jnp.float32),
            pltpu.VMEM((_G, 16), jnp.float32),
            pltpu.VMEM((_G, 16), jnp.float32),
            pltpu.VMEM((16,), jnp.float32),
            pltpu.VMEM_SHARED((n_rows, 16), jnp.float32),
            pltpu.SemaphoreType.DMA,
            pltpu.SemaphoreType.DMA,
            pltpu.SemaphoreType.DMA,
            pltpu.SemaphoreType.DMA,
            pltpu.SemaphoreType.DMA,
            pltpu.SemaphoreType.DMA,
        ],
    )
    def edge_kernel(aug_hbm, idx_hbm, out_hbm, idx0, idx1, zi0, zi1,
                    zj0, zj1, accv, aug_sp, sx0, sx1, si0, si1, sj0, sj1):
        idxb = (idx0, idx1)
        zib = (zi0, zi1)
        zjb = (zj0, zj1)
        sx = (sx0, sx1)
        si = (si0, si1)
        sj = (sj0, sj1)
        wid = lax.axis_index("s") * 2 + lax.axis_index("c")
        row0 = wid * T

        def idx_issue(t, b):
            return pltpu.async_copy(idx_hbm.at[row0 + t], idxb[b], sx[b])

        def idx_wait(t, b):
            pltpu.make_async_copy(idx_hbm.at[row0 + t], idxb[b],
                                  sx[b]).wait()

        def gather_issue(b):
            pltpu.async_copy(aug_sp.at[idxb[b].at[0]], zib[b], si[b])
            pltpu.async_copy(aug_sp.at[idxb[b].at[1]], zjb[b], sj[b])

        def gather_wait(b):
            pltpu.make_async_copy(aug_sp.at[idxb[b].at[0]], zib[b],
                                  si[b]).wait()
            pltpu.make_async_copy(aug_sp.at[idxb[b].at[1]], zjb[b],
                                  sj[b]).wait()

        def compute(b, acc):
            for u in range(_G // 16):
                e = u * 16 + lax.iota(jnp.int32, 16)
                d2 = jnp.full((16,), 1e-8, jnp.float32)
                for d in range(8):
                    col = jnp.full((16,), d, jnp.int32)
                    ai = plsc.load_gather(zib[b], [e, col])
                    aj = plsc.load_gather(zjb[b], [e, col])
                    t_ = ai - aj
                    d2 = d2 + t_ * t_
                col8 = jnp.full((16,), 8, jnp.int32)
                gi = plsc.load_gather(zib[b], [e, col8])
                gj = plsc.load_gather(zjb[b], [e, col8])
                acc = acc + (gi + gj - _sc_sqrt(d2))
            return acc

        @pl.when(lax.axis_index("s") == 0)
        def _():
            pltpu.sync_copy(aug_hbm, aug_sp)

        h0 = idx_issue(0, 0)
        idx_issue(1, 1)
        plsc.subcore_barrier()
        h0.wait()
        gather_issue(0)

        def one_step(t, b, acc):
            gather_wait(b)

            @pl.when(t + 2 < T)
            def _():
                idx_issue(t + 2, b)

            @pl.when(t + 1 < T)
            def _():
                idx_wait(t + 1, 1 - b)
                gather_issue(1 - b)

            return compute(b, acc)

        def pair(k, acc):
            t0 = 2 * k
            acc = one_step(t0, 0, acc)
            acc = one_step(t0 + 1, 1, acc)
            return acc

        acc = lax.fori_loop(0, T // 2, pair,
                            jnp.zeros((16,), jnp.float32))
        accv[...] = acc
        pltpu.sync_copy(accv, out_hbm.at[wid])

    return edge_kernel(aug, idx_rows)


def _tc_body(n_steps, n_edges, seg_ref, z_ref, g_ref, bias_ref, part_ref,
             out_ref, stats):
    i = pl.program_id(0)

    @pl.when(i == 0)
    def _():
        stats[...] = jnp.zeros((_K, 16), jnp.float32)

    seg = seg_ref[...]
    iota_k = lax.broadcasted_iota(jnp.int32, (_K, _B), 0)
    oh = jnp.where(iota_k == seg, 1.0, 0.0).astype(jnp.float32)
    eg = jnp.exp(g_ref[...])
    ones = jnp.ones((_B, 1), jnp.float32)
    zeros6 = jnp.zeros((_B, 6), jnp.float32)
    payload = jnp.concatenate([z_ref[...], ones, eg, zeros6], axis=1)
    stats[...] += lax.dot_general(
        oh, payload, (((1,), (0,)), ((), ())),
        preferred_element_type=jnp.float32,
        precision=lax.Precision.HIGHEST)

    @pl.when(i == n_steps - 1)
    def _():
        st = stats[...]
        counts = st[:, 8:9]
        s_col = st[:, 9:10]
        cm = st[:, 0:8] / jnp.maximum(counts, 1.0)
        cc = cm * cm
        n_col = jnp.sum(cc, axis=1, keepdims=True)
        ones8 = jnp.ones((1, 8), jnp.float32)
        n_row = lax.dot_general(
            ones8, cc, (((1,), (1,)), ((), ())),
            preferred_element_type=jnp.float32,
            precision=lax.Precision.HIGHEST)
        e9 = jnp.where(
            lax.broadcasted_iota(jnp.int32, (1, 16), 1) == 9, 1.0, 0.0
        ).astype(jnp.float32)
        s_row = lax.dot_general(
            e9, st, (((1,), (1,)), ((), ())),
            preferred_element_type=jnp.float32,
            precision=lax.Precision.HIGHEST)
        bias = bias_ref[0, 0]
        total = jnp.float32(0.0)
        for rb in range(_K // 128):
            r0 = rb * 128
            cr = cm[r0:r0 + 128, :]
            g_mat = lax.dot_general(
                cr, cm, (((1,), (1,)), ((), ())),
                preferred_element_type=jnp.float32,
                precision=lax.Precision.HIGHEST)
            d2 = jnp.maximum(
                n_col[r0:r0 + 128, :] + n_row - 2.0 * g_mat, 0.0) + 1e-8
            kx = jnp.exp(bias - jnp.sqrt(d2))
            w = s_col[r0:r0 + 128, :] * s_row
            row_id = r0 + lax.broadcasted_iota(jnp.int32, (128, _K), 0)
            col_id = lax.broadcasted_iota(jnp.int32, (128, _K), 1)
            total += jnp.sum(jnp.where(col_id > row_id, kx * w, 0.0))
        link = (jnp.float32(n_edges) * bias + jnp.sum(part_ref[...]))
        out_ref[...] = jnp.reshape(link - total, (1, 1))


def kernel(latent_z, gamma, bias, segment_ids, sparse_i, sparse_j):
    n = latent_z.shape[0]
    e = sparse_i.shape[0]
    z = latent_z.astype(jnp.float32)
    g = gamma.astype(jnp.float32)

    aug = jnp.concatenate(
        [z, g[:, None], jnp.zeros((n, 7), jnp.float32)], axis=1)
    aug = jnp.concatenate([aug, jnp.zeros((8, 16), jnp.float32)], axis=0)
    grp = _NW * _G
    ep = ((e + grp - 1) // grp) * grp
    pad = ep - e
    t_per_w = ep // grp
    assert t_per_w % 2 == 0 and t_per_w >= 2
    si = jnp.concatenate(
        [sparse_i.astype(jnp.int32), jnp.full((pad,), n, jnp.int32)])
    sj = jnp.concatenate(
        [sparse_j.astype(jnp.int32), jnp.full((pad,), n, jnp.int32)])
    idx_rows = jnp.stack(
        [si.reshape(ep // _G, _G), sj.reshape(ep // _G, _G)], axis=1)

    partials = _sc_edge_call(aug, idx_rows, t_per_w)

    pad_fix = jnp.float32(pad) * jnp.float32(1e-8) ** 0.5

    n_pad = ((n + _B - 1) // _B) * _B
    n_steps = n_pad // _B
    seg_row = jnp.concatenate(
        [segment_ids.astype(jnp.int32),
         jnp.full((n_pad - n,), _K, jnp.int32)]).reshape(1, n_pad)
    z_pad = jnp.concatenate(
        [z, jnp.zeros((n_pad - n, 8), jnp.float32)], axis=0)
    g_pad = jnp.concatenate(
        [g, jnp.zeros((n_pad - n,), jnp.float32)]).reshape(n_pad, 1)
    bias2 = bias.astype(jnp.float32).reshape(1, 1)

    out = pl.pallas_call(
        functools.partial(_tc_body, n_steps, e),
        grid=(n_steps,),
        in_specs=[
            pl.BlockSpec((1, _B), lambda i: (0, i)),
            pl.BlockSpec((_B, 8), lambda i: (i, 0)),
            pl.BlockSpec((_B, 1), lambda i: (i, 0)),
            pl.BlockSpec((1, 1), lambda i: (0, 0)),
            pl.BlockSpec((_NW, 16), lambda i: (0, 0)),
        ],
        out_specs=pl.BlockSpec((1, 1), lambda i: (0, 0)),
        out_shape=jax.ShapeDtypeStruct((1, 1), jnp.float32),
        scratch_shapes=[pltpu.VMEM((_K, 16), jnp.float32)],
    )(seg_row, z_pad, g_pad, bias2, partials)

    return out[0, 0] + pad_fix


# speedup vs baseline: 139.6168x; 1.4828x over previous
import functools

import jax
import jax.numpy as jnp
from jax import lax
from jax.experimental import pallas as pl
from jax.experimental.pallas import tpu as pltpu
from jax.experimental.pallas import tpu_sc as plsc


def _sc_sqrt(x):
    xi = plsc.bitcast(x, jnp.int32)
    ri = jnp.int32(0x5F3759DF) - lax.shift_right_logical(xi, jnp.ones_like(xi))
    r = plsc.bitcast(ri, jnp.float32)
    half_x = 0.5 * x
    for _ in range(3):
        r = r * (1.5 - half_x * r * r)
    return x * r


_K = 1024
_NW = 32
_G = 128


def _sc_call(aug, seg, zeros_tbl, idx_rows, T, n_real, C):
    n_rows = aug.shape[0]
    mesh = plsc.VectorSubcoreMesh(core_axis_name="c", subcore_axis_name="s")

    @functools.partial(
        pl.kernel,
        mesh=mesh,
        compiler_params=pltpu.CompilerParams(
            needs_layout_passes=False, use_tc_tiling_on_sc=False),
        out_type=(
            jax.ShapeDtypeStruct((_NW, 16), jnp.float32),
            jax.ShapeDtypeStruct((2, _K, 16), jnp.float32),
            jax.ShapeDtypeStruct((_NW, _K), jnp.float32),
        ),
        scratch_types=[
            pltpu.VMEM((2, _G), jnp.int32),
            pltpu.VMEM((2, _G), jnp.int32),
            pltpu.VMEM((_G, 16), jnp.float32),
            pltpu.VMEM((_G, 16), jnp.float32),
            pltpu.VMEM((_G, 16), jnp.float32),
            pltpu.VMEM((_G, 16), jnp.float32),
            pltpu.VMEM((16,), jnp.float32),
            pltpu.VMEM((128, 16), jnp.float32),
            pltpu.VMEM((128,), jnp.int32),
            pltpu.VMEM((_K,), jnp.float32),
            pltpu.VMEM_SHARED((n_rows, 16), jnp.float32),
            pltpu.VMEM_SHARED((_K, 16), jnp.float32),
            pltpu.SemaphoreType.DMA,
            pltpu.SemaphoreType.DMA,
            pltpu.SemaphoreType.DMA,
            pltpu.SemaphoreType.DMA,
            pltpu.SemaphoreType.DMA,
            pltpu.SemaphoreType.DMA,
        ],
    )
    def sc_kernel(aug_hbm, seg_hbm, zeros_hbm, idx_hbm,
                  out_hbm, stats_hbm, se_hbm,
                  idx0, idx1, zi0, zi1, zj0, zj1, accv, nb, segb, seb,
                  aug_sp, stats_sp, sx0, sx1, si0, si1, sj0, sj1):
        idxb = (idx0, idx1)
        zib = (zi0, zi1)
        zjb = (zj0, zj1)
        sx = (sx0, sx1)
        si = (si0, si1)
        sj = (sj0, sj1)
        cid = lax.axis_index("c")
        sid = lax.axis_index("s")
        wid = sid * 2 + cid
        row0 = wid * T

        def idx_issue(t, b):
            return pltpu.async_copy(idx_hbm.at[row0 + t], idxb[b], sx[b])

        def idx_wait(t, b):
            pltpu.make_async_copy(idx_hbm.at[row0 + t], idxb[b],
                                  sx[b]).wait()

        def gather_issue(b):
            pltpu.async_copy(aug_sp.at[idxb[b].at[0]], zib[b], si[b])
            pltpu.async_copy(aug_sp.at[idxb[b].at[1]], zjb[b], sj[b])

        def gather_wait(b):
            pltpu.make_async_copy(aug_sp.at[idxb[b].at[0]], zib[b],
                                  si[b]).wait()
            pltpu.make_async_copy(aug_sp.at[idxb[b].at[1]], zjb[b],
                                  sj[b]).wait()

        def compute(b, acc):
            for u in range(_G // 16):
                e = u * 16 + lax.iota(jnp.int32, 16)
                d2 = jnp.full((16,), 1e-8, jnp.float32)
                for d in range(8):
                    col = jnp.full((16,), d, jnp.int32)
                    ai = plsc.load_gather(zib[b], [e, col])
                    aj = plsc.load_gather(zjb[b], [e, col])
                    t_ = ai - aj
                    d2 = d2 + t_ * t_
                col8 = jnp.full((16,), 8, jnp.int32)
                gi = plsc.load_gather(zib[b], [e, col8])
                gj = plsc.load_gather(zjb[b], [e, col8])
                acc = acc + (gi + gj - _sc_sqrt(d2))
            return acc

        @pl.when(sid == 0)
        def _():
            pltpu.sync_copy(aug_hbm, aug_sp)
            pltpu.sync_copy(zeros_hbm, stats_sp)

        h0 = idx_issue(0, 0)
        idx_issue(1, 1)
        for i in range(_K // 16):
            seb[pl.ds(i * 16, 16)] = jnp.zeros((16,), jnp.float32)
        plsc.subcore_barrier()
        h0.wait()
        gather_issue(0)

        node_base = wid * (C * 128)

        def stats_chunk(c, carry):
            node0 = node_base + c * 128
            pltpu.sync_copy(aug_sp.at[pl.ds(node0, 128)], nb)
            pltpu.sync_copy(seg_hbm.at[pl.ds(node0, 128)], segb)
            pltpu.sync_copy(nb, stats_sp.at[segb], add=True)
            for u in range(8):
                lanes = u * 16 + lax.iota(jnp.int32, 16)
                gcol = plsc.load_gather(nb, [lanes, jnp.full((16,), 8,
                                                            jnp.int32)])
                ev = jnp.exp(gcol)
                segv = segb[pl.ds(u * 16, 16)]
                nid = node0 + u * 16 + lax.iota(jnp.int32, 16)
                plsc.addupdate_scatter(seb, [segv], ev, mask=nid < n_real)
            return carry

        lax.fori_loop(0, C, stats_chunk, 0)
        plsc.subcore_barrier()
        pltpu.sync_copy(seb, se_hbm.at[wid])

        @pl.when(sid == 0)
        def _():
            pltpu.sync_copy(stats_sp, stats_hbm.at[cid])

        def one_step(t, b, acc):
            gather_wait(b)

            @pl.when(t + 2 < T)
            def _():
                idx_issue(t + 2, b)

            @pl.when(t + 1 < T)
            def _():
                idx_wait(t + 1, 1 - b)
                gather_issue(1 - b)

            return compute(b, acc)

        def pair(k, acc):
            t0 = 2 * k
            acc = one_step(t0, 0, acc)
            acc = one_step(t0 + 1, 1, acc)
            return acc

        acc = lax.fori_loop(0, T // 2, pair,
                            jnp.zeros((16,), jnp.float32))
        accv[...] = acc
        pltpu.sync_copy(accv, out_hbm.at[wid])

    return sc_kernel(aug, seg, zeros_tbl, idx_rows)


def _tc_body(n_edges, st2_ref, se_ref, bias_ref, part_ref, out_ref):
    hi = lax.Precision.HIGHEST
    st = st2_ref[0:_K, :] + st2_ref[_K:2 * _K, :]
    counts = st[:, 9:10]
    cm = st[:, 0:8] / jnp.maximum(counts, 1.0)
    s_row = jnp.sum(se_ref[...], axis=0, keepdims=True)
    ident = jnp.where(
        lax.broadcasted_iota(jnp.int32, (_K, _K), 0)
        == lax.broadcasted_iota(jnp.int32, (_K, _K), 1), 1.0, 0.0
    ).astype(jnp.float32)
    s_col = lax.dot_general(ident, s_row, (((1,), (1,)), ((), ())),
                            preferred_element_type=jnp.float32,
                            precision=hi)
    cc = cm * cm
    n_col = jnp.sum(cc, axis=1, keepdims=True)
    ones8 = jnp.ones((1, 8), jnp.float32)
    n_row = lax.dot_general(ones8, cc, (((1,), (1,)), ((), ())),
                            preferred_element_type=jnp.float32,
                            precision=hi)
    bias = bias_ref[0, 0]
    total = jnp.float32(0.0)
    for rb in range(_K // 128):
        r0 = rb * 128
        cr = cm[r0:r0 + 128, :]
        g_mat = lax.dot_general(cr, cm, (((1,), (1,)), ((), ())),
                                preferred_element_type=jnp.float32,
                                precision=hi)
        d2 = jnp.maximum(
            n_col[r0:r0 + 128, :] + n_row - 2.0 * g_mat, 0.0) + 1e-8
        kx = jnp.exp(bias - jnp.sqrt(d2))
        w = s_col[r0:r0 + 128, :] * s_row
        row_id = r0 + lax.broadcasted_iota(jnp.int32, (128, _K), 0)
        col_id = lax.broadcasted_iota(jnp.int32, (128, _K), 1)
        total += jnp.sum(jnp.where(col_id > row_id, kx * w, 0.0))
    link = jnp.float32(n_edges) * bias + jnp.sum(part_ref[...])
    out_ref[...] = jnp.reshape(link - total, (1, 1))


def kernel(latent_z, gamma, bias, segment_ids, sparse_i, sparse_j):
    n = latent_z.shape[0]
    e = sparse_i.shape[0]
    z = latent_z.astype(jnp.float32)
    g = gamma.astype(jnp.float32)

    node_grp = _NW * 128
    c_chunks = (n + node_grp - 1) // node_grp
    n_pad = c_chunks * node_grp
    aug = jnp.concatenate(
        [z, g[:, None], jnp.ones((n, 1), jnp.float32),
         jnp.zeros((n, 6), jnp.float32)], axis=1)
    aug = jnp.concatenate(
        [aug, jnp.zeros((n_pad - n, 16), jnp.float32)], axis=0)
    seg = jnp.concatenate(
        [segment_ids.astype(jnp.int32), jnp.zeros((n_pad - n,), jnp.int32)])
    zeros_tbl = jnp.zeros((_K, 16), jnp.float32)

    grp = _NW * _G
    ep = ((e + grp - 1) // grp) * grp
    pad = ep - e
    t_per_w = ep // grp
    assert t_per_w % 2 == 0 and t_per_w >= 2
    si = jnp.concatenate(
        [sparse_i.astype(jnp.int32), jnp.full((pad,), n, jnp.int32)])
    sj = jnp.concatenate(
        [sparse_j.astype(jnp.int32), jnp.full((pad,), n, jnp.int32)])
    idx_rows = jnp.stack(
        [si.reshape(ep // _G, _G), sj.reshape(ep // _G, _G)], axis=1)

    partials, stats2, se = _sc_call(
        aug, seg, zeros_tbl, idx_rows, t_per_w, n, c_chunks)

    pad_fix = jnp.float32(pad) * jnp.float32(1e-8) ** 0.5

    out = pl.pallas_call(
        functools.partial(_tc_body, e),
        grid=(1,),
        in_specs=[
            pl.BlockSpec((2 * _K, 16), lambda i: (0, 0)),
            pl.BlockSpec((_NW, _K), lambda i: (0, 0)),
            pl.BlockSpec((1, 1), lambda i: (0, 0)),
            pl.BlockSpec((_NW, 16), lambda i: (0, 0)),
        ],
        out_specs=pl.BlockSpec((1, 1), lambda i: (0, 0)),
        out_shape=jax.ShapeDtypeStruct((1, 1), jnp.float32),
    )(stats2.reshape(2 * _K, 16), se, bias.astype(jnp.float32).reshape(1, 1),
      partials)

    return out[0, 0] + pad_fix


# speedup vs baseline: 149.0432x; 1.0675x over previous
import functools

import jax
import jax.numpy as jnp
from jax import lax
from jax.experimental import pallas as pl
from jax.experimental.pallas import tpu as pltpu
from jax.experimental.pallas import tpu_sc as plsc


def _sc_sqrt(x):
    xi = plsc.bitcast(x, jnp.int32)
    ri = jnp.int32(0x5F3759DF) - lax.shift_right_logical(xi, jnp.ones_like(xi))
    r = plsc.bitcast(ri, jnp.float32)
    half_x = 0.5 * x
    for _ in range(3):
        r = r * (1.5 - half_x * r * r)
    return x * r


_K = 1024
_NW = 32
_G = 128


def _sc_call(aug, pk, seg, zeros_tbl, si2, sj2, T, n_real, C):
    n_rows = pk.shape[0]
    mesh = plsc.VectorSubcoreMesh(core_axis_name="c", subcore_axis_name="s")

    @functools.partial(
        pl.kernel,
        mesh=mesh,
        compiler_params=pltpu.CompilerParams(
            needs_layout_passes=False, use_tc_tiling_on_sc=False),
        out_type=(
            jax.ShapeDtypeStruct((_NW, 16), jnp.float32),
            jax.ShapeDtypeStruct((2, _K, 16), jnp.float32),
            jax.ShapeDtypeStruct((_NW, _K), jnp.float32),
        ),
        scratch_types=[
            pltpu.VMEM((2, _G), jnp.int32),
            pltpu.VMEM((2, _G), jnp.int32),
            pltpu.VMEM((_G, 8), jnp.int32),
            pltpu.VMEM((_G, 8), jnp.int32),
            pltpu.VMEM((_G, 8), jnp.int32),
            pltpu.VMEM((_G, 8), jnp.int32),
            pltpu.VMEM((16,), jnp.float32),
            pltpu.VMEM((128, 16), jnp.float32),
            pltpu.VMEM((128,), jnp.int32),
            pltpu.VMEM((_K,), jnp.float32),
            pltpu.VMEM_SHARED((n_rows, 8), jnp.int32),
            pltpu.VMEM_SHARED((_K, 16), jnp.float32),
            pltpu.SemaphoreType.DMA,
            pltpu.SemaphoreType.DMA,
            pltpu.SemaphoreType.DMA,
            pltpu.SemaphoreType.DMA,
            pltpu.SemaphoreType.DMA,
            pltpu.SemaphoreType.DMA,
        ],
    )
    def sc_kernel(aug_hbm, pk_hbm, seg_hbm, zeros_hbm, si2_hbm, sj2_hbm,
                  out_hbm, stats_hbm, se_hbm,
                  idx0, idx1, zi0, zi1, zj0, zj1, accv, nb, segb, seb,
                  pk_sp, stats_sp, sx0, sx1, si0, si1, sj0, sj1):
        idxb = (idx0, idx1)
        zib = (zi0, zi1)
        zjb = (zj0, zj1)
        sx = (sx0, sx1)
        si = (si0, si1)
        sj = (sj0, sj1)
        cid = lax.axis_index("c")
        sid = lax.axis_index("s")
        wid = sid * 2 + cid
        row0 = wid * T

        def idx_issue(t, b):
            pltpu.async_copy(si2_hbm.at[row0 + t], idxb[b].at[0], sx[b])
            pltpu.async_copy(sj2_hbm.at[row0 + t], idxb[b].at[1], sx[b])

        def idx_wait(t, b):
            pltpu.make_async_copy(si2_hbm.at[row0 + t], idxb[b].at[0],
                                  sx[b]).wait()
            pltpu.make_async_copy(sj2_hbm.at[row0 + t], idxb[b].at[1],
                                  sx[b]).wait()

        def gather_issue(b):
            pltpu.async_copy(pk_sp.at[idxb[b].at[0]], zib[b], si[b])
            pltpu.async_copy(pk_sp.at[idxb[b].at[1]], zjb[b], sj[b])

        def gather_wait(b):
            pltpu.make_async_copy(pk_sp.at[idxb[b].at[0]], zib[b],
                                  si[b]).wait()
            pltpu.make_async_copy(pk_sp.at[idxb[b].at[1]], zjb[b],
                                  sj[b]).wait()

        def compute(b, acc):
            for u in range(_G // 16):
                e = u * 16 + lax.iota(jnp.int32, 16)
                d2 = jnp.full((16,), 1e-8, jnp.float32)
                for d in range(7):
                    col = jnp.full((16,), d, jnp.int32)
                    ai = plsc.bitcast(plsc.load_gather(zib[b], [e, col]),
                                      jnp.float32)
                    aj = plsc.bitcast(plsc.load_gather(zjb[b], [e, col]),
                                      jnp.float32)
                    t_ = ai - aj
                    d2 = d2 + t_ * t_
                col7 = jnp.full((16,), 7, jnp.int32)
                wi = plsc.bitcast(plsc.load_gather(zib[b], [e, col7]),
                                  jnp.bfloat16)
                wj = plsc.bitcast(plsc.load_gather(zjb[b], [e, col7]),
                                  jnp.bfloat16)
                z7i, gi = plsc.unpack(wi, format=plsc.PackFormat.INTERLEAVED)
                z7j, gj = plsc.unpack(wj, format=plsc.PackFormat.INTERLEAVED)
                t7 = z7i - z7j
                d2 = d2 + t7 * t7
                acc = acc + (gi + gj - _sc_sqrt(d2))
            return acc

        @pl.when(sid == 0)
        def _():
            pltpu.sync_copy(pk_hbm, pk_sp)
            pltpu.sync_copy(zeros_hbm, stats_sp)

        idx_issue(0, 0)
        idx_issue(1, 1)
        for i in range(_K // 16):
            seb[pl.ds(i * 16, 16)] = jnp.zeros((16,), jnp.float32)
        plsc.subcore_barrier()
        idx_wait(0, 0)
        gather_issue(0)

        node_base = wid * (C * 128)

        def stats_chunk(c, carry):
            node0 = node_base + c * 128
            pltpu.sync_copy(aug_hbm.at[pl.ds(node0, 128)], nb)
            pltpu.sync_copy(seg_hbm.at[pl.ds(node0, 128)], segb)
            pltpu.sync_copy(nb, stats_sp.at[segb], add=True)
            for u in range(8):
                lanes = u * 16 + lax.iota(jnp.int32, 16)
                gcol = plsc.load_gather(nb, [lanes, jnp.full((16,), 8,
                                                            jnp.int32)])
                ev = jnp.exp(gcol)
                segv = segb[pl.ds(u * 16, 16)]
                nid = node0 + u * 16 + lax.iota(jnp.int32, 16)
                plsc.addupdate_scatter(seb, [segv], ev, mask=nid < n_real)
            return carry

        lax.fori_loop(0, C, stats_chunk, 0)
        plsc.subcore_barrier()
        pltpu.sync_copy(seb, se_hbm.at[wid])

        @pl.when(sid == 0)
        def _():
            pltpu.sync_copy(stats_sp, stats_hbm.at[cid])

        def one_step(t, b, acc):
            gather_wait(b)

            @pl.when(t + 2 < T)
            def _():
                idx_issue(t + 2, b)

            @pl.when(t + 1 < T)
            def _():
                idx_wait(t + 1, 1 - b)
                gather_issue(1 - b)

            return compute(b, acc)

        def pair(k, acc):
            t0 = 2 * k
            acc = one_step(t0, 0, acc)
            acc = one_step(t0 + 1, 1, acc)
            return acc

        acc = lax.fori_loop(0, T // 2, pair,
                            jnp.zeros((16,), jnp.float32))
        accv[...] = acc
        pltpu.sync_copy(accv, out_hbm.at[wid])

    return sc_kernel(aug, pk, seg, zeros_tbl, si2, sj2)


def _tc_body(n_edges, st2_ref, se_ref, bias_ref, part_ref, out_ref):
    hi = lax.Precision.HIGHEST
    st = st2_ref[0:_K, :] + st2_ref[_K:2 * _K, :]
    counts = st[:, 9:10]
    cm = st[:, 0:8] / jnp.maximum(counts, 1.0)
    s_row = jnp.sum(se_ref[...], axis=0, keepdims=True)
    ident = jnp.where(
        lax.broadcasted_iota(jnp.int32, (_K, _K), 0)
        == lax.broadcasted_iota(jnp.int32, (_K, _K), 1), 1.0, 0.0
    ).astype(jnp.float32)
    s_col = lax.dot_general(ident, s_row, (((1,), (1,)), ((), ())),
                            preferred_element_type=jnp.float32,
                            precision=hi)
    cc = cm * cm
    n_col = jnp.sum(cc, axis=1, keepdims=True)
    ones8 = jnp.ones((1, 8), jnp.float32)
    n_row = lax.dot_general(ones8, cc, (((1,), (1,)), ((), ())),
                            preferred_element_type=jnp.float32,
                            precision=hi)
    bias = bias_ref[0, 0]
    total = jnp.float32(0.0)
    for rb in range(_K // 128):
        r0 = rb * 128
        cr = cm[r0:r0 + 128, :]
        g_mat = lax.dot_general(cr, cm, (((1,), (1,)), ((), ())),
                                preferred_element_type=jnp.float32,
                                precision=hi)
        d2 = jnp.maximum(
            n_col[r0:r0 + 128, :] + n_row - 2.0 * g_mat, 0.0) + 1e-8
        kx = jnp.exp(bias - jnp.sqrt(d2))
        w = s_col[r0:r0 + 128, :] * s_row
        row_id = r0 + lax.broadcasted_iota(jnp.int32, (128, _K), 0)
        col_id = lax.broadcasted_iota(jnp.int32, (128, _K), 1)
        total += jnp.sum(jnp.where(col_id > row_id, kx * w, 0.0))
    link = jnp.float32(n_edges) * bias + jnp.sum(part_ref[...])
    out_ref[...] = jnp.reshape(link - total, (1, 1))


def kernel(latent_z, gamma, bias, segment_ids, sparse_i, sparse_j):
    n = latent_z.shape[0]
    e = sparse_i.shape[0]
    z = latent_z.astype(jnp.float32)
    g = gamma.astype(jnp.float32)

    node_grp = _NW * 128
    c_chunks = (n + node_grp - 1) // node_grp
    n_pad = c_chunks * node_grp
    aug = jnp.concatenate(
        [z, g[:, None], jnp.ones((n, 1), jnp.float32),
         jnp.zeros((n, 6), jnp.float32)], axis=1)
    aug = jnp.concatenate(
        [aug, jnp.zeros((n_pad - n, 16), jnp.float32)], axis=0)
    seg = jnp.concatenate(
        [segment_ids.astype(jnp.int32), jnp.zeros((n_pad - n,), jnp.int32)])
    zeros_tbl = jnp.zeros((_K, 16), jnp.float32)

    zw = lax.bitcast_convert_type(z[:, 0:7], jnp.int32)
    lo = lax.bitcast_convert_type(
        z[:, 7].astype(jnp.bfloat16), jnp.uint16).astype(jnp.uint32)
    hi = lax.bitcast_convert_type(
        g.astype(jnp.bfloat16), jnp.uint16).astype(jnp.uint32)
    w7 = lax.bitcast_convert_type(lo | (hi << 16), jnp.int32)
    pk = jnp.concatenate([zw, w7[:, None]], axis=1)
    pk = jnp.concatenate([pk, jnp.zeros((8, 8), jnp.int32)], axis=0)

    grp = _NW * _G
    ep = ((e + grp - 1) // grp) * grp
    pad = ep - e
    t_per_w = ep // grp
    assert t_per_w % 2 == 0 and t_per_w >= 2
    si2 = jnp.concatenate(
        [sparse_i.astype(jnp.int32),
         jnp.full((pad,), n, jnp.int32)]).reshape(ep // _G, _G)
    sj2 = jnp.concatenate(
        [sparse_j.astype(jnp.int32),
         jnp.full((pad,), n, jnp.int32)]).reshape(ep // _G, _G)

    partials, stats2, se = _sc_call(
        aug, pk, seg, zeros_tbl, si2, sj2, t_per_w, n, c_chunks)

    pad_fix = jnp.float32(pad) * jnp.float32(1e-8) ** 0.5

    out = pl.pallas_call(
        functools.partial(_tc_body, e),
        grid=(1,),
        in_specs=[
            pl.BlockSpec((2 * _K, 16), lambda i: (0, 0)),
            pl.BlockSpec((_NW, _K), lambda i: (0, 0)),
            pl.BlockSpec((1, 1), lambda i: (0, 0)),
            pl.BlockSpec((_NW, 16), lambda i: (0, 0)),
        ],
        out_specs=pl.BlockSpec((1, 1), lambda i: (0, 0)),
        out_shape=jax.ShapeDtypeStruct((1, 1), jnp.float32),
    )(stats2.reshape(2 * _K, 16), se, bias.astype(jnp.float32).reshape(1, 1),
      partials)

    return out[0, 0] + pad_fix
